# Initial kernel scaffold; baseline (speedup 1.0000x reference)
import jax
import jax.numpy as jnp
from jax.experimental import pallas as pl


def kernel(X, edge_index, edge_weight, attention, W_i, W_f, W_c, W_o, b_i, b_f, b_c, b_o):
    raise NotImplementedError("write your pallas kernel here")


# speedup vs baseline: 8.9611x; 8.9611x over previous
import functools

import jax
import jax.numpy as jnp
from jax import lax
from jax.experimental import pallas as pl
from jax.experimental.pallas import tpu as pltpu
from jax.experimental.pallas import tpu_sc as plsc

N = 10000
E = 320000

_BCAST_DNUMS = lax.GatherDimensionNumbers(
    offset_dims=(), collapsed_slice_dims=(0,), start_index_map=(0,))


def _lane_bcast(vec, lane):
    idx = jnp.full((16, 1), lane, jnp.int32)
    return lax.gather(vec, idx, _BCAST_DNUMS, slice_sizes=(1,),
                      mode=lax.GatherScatterMode.PROMISE_IN_BOUNDS)
F_IN = 128
F_OUT = 128
PERIODS = 12

NC, NS = 2, 16
NTILES = NC * NS
CHUNK = 128
SUP = 8
N_PAD = 10240
ROWS_PER_TILE = N_PAD // NS
WB = 128
CPT = 80
NSUP = CPT // SUP
E_PAD = NTILES * CPT * CHUNK
BN = 1000


F = 128
CB = F // 16


@functools.lru_cache(maxsize=None)
def _sc_prop_fn(P, G):
    mesh = plsc.VectorSubcoreMesh(core_axis_name="c", subcore_axis_name="s",
                                  num_cores=NC, num_subcores=NS)

    @functools.partial(
        pl.kernel,
        out_type=jax.ShapeDtypeStruct((P, NC, N_PAD, F), jnp.float32),
        mesh=mesh,
        scratch_types=[
            [pltpu.VMEM((SUP, 2, CHUNK), jnp.int32)] * 2,
            [pltpu.VMEM((SUP * CHUNK,), jnp.float32)] * 2,
            [pltpu.VMEM((CHUNK,), jnp.int32)] * 2,
            [pltpu.VMEM((CHUNK, F), jnp.float32)] * 2,
            pltpu.VMEM_SHARED((N_PAD, F), jnp.float32),
            [pltpu.SemaphoreType.DMA] * 2,
            [pltpu.SemaphoreType.DMA] * 2,
        ],
    )
    def body(table_h, pk_h, w_h, zwb_h, out_h, pk, wsup, aidx, rows, accum,
             psem, gsem):
        c = lax.axis_index("c")
        s = lax.axis_index("s")
        tl = c * NS + s
        row0 = s * ROWS_PER_TILE

        def sup_start(su, b):
            base = tl * NSUP + su
            pltpu.async_copy(pk_h.at[pl.ds(base * SUP, SUP)], pk[b], psem[b])
            pltpu.async_copy(w_h.at[pl.ds(base * SUP * CHUNK, SUP * CHUNK)],
                             wsup[b], psem[b])

        def sup_wait(su, b):
            base = tl * NSUP + su
            pltpu.make_async_copy(pk_h.at[pl.ds(base * SUP, SUP)], pk[b],
                                  psem[b]).wait()
            pltpu.make_async_copy(w_h.at[pl.ds(base * SUP * CHUNK, SUP * CHUNK)],
                                  wsup[b], psem[b]).wait()

        def gather_start(off, u, sb, rb):
            if not G:
                return
            for g in range(CHUNK // 16):
                sl = pl.ds(g * 16, 16)
                aidx[rb][sl] = pk[sb][u, 0, sl] + off
            pltpu.async_copy(table_h.at[aidx[rb]], rows[rb], gsem[rb])

        def gather_wait(rb):
            if G:
                pltpu.make_async_copy(table_h.at[aidx[rb]], rows[rb],
                                      gsem[rb]).wait()

        def scale(u, sb, rb):
            def group(g, carry3):
                wvec = wsup[sb][pl.ds(u * CHUNK + g * 16, 16)]
                for e in range(16):
                    col = g * 16 + e
                    wb = _lane_bcast(wvec, e)
                    for cb in range(CB):
                        sl = pl.ds(cb * 16, 16)
                        if G:
                            rows[rb][col, sl] = rows[rb][col, sl] * wb
                        else:
                            rows[rb][col, sl] = wvec * 0.0 + wb
                return carry3
            lax.fori_loop(0, CHUNK // 16, group, 0)

        def period(p, carry):
            pltpu.sync_copy(zwb_h, accum.at[pl.ds(row0, ROWS_PER_TILE)])
            plsc.subcore_barrier()
            off = p * N

            sup_start(0, 0)
            sup_wait(0, 0)
            gather_start(off, 0, 0, 0)
            sup_start(1, 1)

            def super_iter(ss, carry2):
              for sb in range(2):
                su = ss * 2 + sb
                for u in range(SUP):
                    rb = u % 2
                    nrb = 1 - rb
                    if u + 1 < SUP:
                        gather_start(off, u + 1, sb, nrb)
                    else:
                        @pl.when(su + 1 < NSUP)
                        def _():
                            sup_wait(su + 1, 1 - sb)
                            gather_start(off, 0, 1 - sb, nrb)
                    gather_wait(rb)
                    scale(u, sb, rb)
                    pltpu.sync_copy(rows[rb], accum.at[pk[sb].at[u, 1]],
                                    add=True)

                @pl.when(su + 2 < NSUP)
                def _():
                    sup_start(su + 2, sb)
              return carry2

            lax.fori_loop(0, NSUP // 2, super_iter, 0)
            plsc.subcore_barrier()
            pltpu.sync_copy(accum.at[pl.ds(row0, ROWS_PER_TILE)],
                            out_h.at[p, c, pl.ds(row0, ROWS_PER_TILE)])
            plsc.subcore_barrier()
            return carry

        lax.fori_loop(0, P, period, 0)

    return body


def _sc_prop(P, table, packed, w1d):
    zwb = jnp.zeros((ROWS_PER_TILE, F), jnp.float32)
    if table is None:
        dummy = jnp.zeros((8, F), jnp.float32)
        out = _sc_prop_fn(P, False)(dummy, packed, w1d, zwb)
    else:
        out = _sc_prop_fn(P, True)(table, packed, w1d, zwb)
    return out[:, :, :N, :]


def _dinv_body(degp_ref, dinv_ref):
    deg = degp_ref[0] + degp_ref[1] + 1.0
    dinv_ref[...] = lax.rsqrt(deg)


def _tc_dinv(degp):
    return pl.pallas_call(
        _dinv_body,
        grid=(N // BN,),
        in_specs=[pl.BlockSpec((2, BN, F), lambda nb: (0, nb, 0))],
        out_specs=pl.BlockSpec((BN, F), lambda nb: (nb, 0)),
        out_shape=jax.ShapeDtypeStruct((N, F), jnp.float32),
    )(degp)


def _xs_body(xt_ref, dinv_ref, xs_ref):
    xs_ref[0] = xt_ref[0] * dinv_ref[:, 0:1]


def _tc_xs(X_t, dinv128):
    return pl.pallas_call(
        _xs_body,
        grid=(PERIODS, N // BN),
        in_specs=[
            pl.BlockSpec((1, BN, F_IN), lambda p, nb: (p, nb, 0)),
            pl.BlockSpec((BN, F), lambda p, nb: (nb, 0)),
        ],
        out_specs=pl.BlockSpec((1, BN, F_IN), lambda p, nb: (p, nb, 0)),
        out_shape=jax.ShapeDtypeStruct((PERIODS, N, F_IN), jnp.float32),
    )(X_t, dinv128)


def _pxw_body(aggx_ref, xs_ref, dinv_ref, rp_ref, wx_ref, ball_ref, pxw_ref):
    dv = dinv_ref[:, 0:1]
    r = dv * (rp_ref[0, :, 0:1] + rp_ref[1, :, 0:1] + dv)
    px = dv * (aggx_ref[0, 0] + aggx_ref[0, 1] + xs_ref[0])
    pxw_ref[0] = (jnp.dot(px, wx_ref[...], preferred_element_type=jnp.float32)
                  + r * ball_ref[...])


def _tc_pxw(aggX, Xs, dinv128, rparts, Wx, b_all):
    grid = (PERIODS, N // BN)
    return pl.pallas_call(
        _pxw_body,
        grid=grid,
        in_specs=[
            pl.BlockSpec((1, 2, BN, F_IN), lambda p, nb: (p, 0, nb, 0)),
            pl.BlockSpec((1, BN, F_IN), lambda p, nb: (p, nb, 0)),
            pl.BlockSpec((BN, F), lambda p, nb: (nb, 0)),
            pl.BlockSpec((2, BN, F), lambda p, nb: (0, nb, 0)),
            pl.BlockSpec((F_IN, 4 * F_OUT), lambda p, nb: (0, 0)),
            pl.BlockSpec((1, 4 * F_OUT), lambda p, nb: (0, 0)),
        ],
        out_specs=pl.BlockSpec((1, BN, 4 * F_OUT), lambda p, nb: (p, nb, 0)),
        out_shape=jax.ShapeDtypeStruct((PERIODS, N, 4 * F_OUT), jnp.float32),
    )(aggX, Xs, dinv128, rparts, Wx, b_all)


def _cell_body(p, pxw_ref, aggh_ref, hsprev_ref, dinv_ref, wh_ref, cprev_ref,
               haccprev_ref, att_ref, c_ref, hs_ref, hacc_ref):
    dv = dinv_ref[:, 0:1]
    ph = dv * (aggh_ref[0] + aggh_ref[1] + hsprev_ref[...])
    g_all = pxw_ref[...] + jnp.dot(ph, wh_ref[...],
                                   preferred_element_type=jnp.float32)
    i = jax.nn.sigmoid(g_all[:, 0:F_OUT])
    f = jax.nn.sigmoid(g_all[:, F_OUT:2 * F_OUT])
    g = jnp.tanh(g_all[:, 2 * F_OUT:3 * F_OUT])
    o = jax.nn.sigmoid(g_all[:, 3 * F_OUT:4 * F_OUT])
    c_new = f * cprev_ref[...] + i * g
    h = o * jnp.tanh(c_new)
    a = att_ref[0]
    ex = jnp.exp(a - jnp.max(a))
    prob = ex[p] / jnp.sum(ex)
    c_ref[...] = c_new
    hs_ref[...] = dv * h
    hacc_ref[...] = haccprev_ref[...] + prob * h


def _tc_cell(p, pxw_p, aggH, Hs_prev, dinv128, Wh, C_prev, Hacc_prev, att2):
    grid = (N // BN,)
    return pl.pallas_call(
        functools.partial(_cell_body, p),
        grid=grid,
        in_specs=[
            pl.BlockSpec((BN, 4 * F_OUT), lambda nb: (nb, 0)),
            pl.BlockSpec((2, BN, F_OUT), lambda nb: (0, nb, 0)),
            pl.BlockSpec((BN, F_OUT), lambda nb: (nb, 0)),
            pl.BlockSpec((BN, F), lambda nb: (nb, 0)),
            pl.BlockSpec((F_OUT, 4 * F_OUT), lambda nb: (0, 0)),
            pl.BlockSpec((BN, F_OUT), lambda nb: (nb, 0)),
            pl.BlockSpec((BN, F_OUT), lambda nb: (nb, 0)),
            pl.BlockSpec((1, PERIODS), lambda nb: (0, 0)),
        ],
        out_specs=[
            pl.BlockSpec((BN, F_OUT), lambda nb: (nb, 0)),
            pl.BlockSpec((BN, F_OUT), lambda nb: (nb, 0)),
            pl.BlockSpec((BN, F_OUT), lambda nb: (nb, 0)),
        ],
        out_shape=[
            jax.ShapeDtypeStruct((N, F_OUT), jnp.float32),
            jax.ShapeDtypeStruct((N, F_OUT), jnp.float32),
            jax.ShapeDtypeStruct((N, F_OUT), jnp.float32),
        ],
    )(pxw_p, aggH, Hs_prev, dinv128, Wh, C_prev, Hacc_prev, att2)


def kernel(X, edge_index, edge_weight, attention,
           W_i, W_f, W_c, W_o, b_i, b_f, b_c, b_o):
    pad = E_PAD - E
    src = jnp.concatenate([edge_index[0], jnp.zeros((pad,), jnp.int32)])
    dst = jnp.concatenate([edge_index[1], jnp.zeros((pad,), jnp.int32)])
    w = jnp.concatenate([edge_weight, jnp.zeros((pad,), jnp.float32)])
    packed = jnp.stack([src.reshape(-1, CHUNK), dst.reshape(-1, CHUNK)],
                       axis=1)

    Wx = jnp.concatenate([W_i[:F_IN], W_f[:F_IN], W_c[:F_IN], W_o[:F_IN]], axis=1)
    Wh = jnp.concatenate([W_i[F_IN:], W_f[F_IN:], W_c[F_IN:], W_o[F_IN:]], axis=1)
    b_all = jnp.concatenate([b_i, b_f, b_c, b_o]).reshape(1, 4 * F_OUT)
    X_t = jnp.transpose(X, (2, 0, 1))
    att2 = attention.reshape(1, PERIODS)

    degp = _sc_prop(1, None, packed, w).reshape(NC, N, F)

    dinv128 = _tc_dinv(degp)
    Xs = _tc_xs(X_t, dinv128)

    rparts = _sc_prop(1, dinv128, packed, w).reshape(NC, N, F)

    aggX = _sc_prop(PERIODS, Xs.reshape(PERIODS * N, F_IN), packed, w)

    PXW = _tc_pxw(aggX, Xs, dinv128, rparts, Wx, b_all)

    zeros_nf = jnp.zeros((N, F_OUT), jnp.float32)
    zeros_agg = jnp.zeros((NC, N, F_OUT), jnp.float32)
    C = zeros_nf
    Hacc = zeros_nf
    Hs = zeros_nf
    aggH = zeros_agg
    for p in range(PERIODS):
        C, Hs, Hacc = _tc_cell(p, PXW[p], aggH, Hs, dinv128, Wh, C, Hacc, att2)
        if p + 1 < PERIODS:
            aggH = _sc_prop(1, Hs, packed, w).reshape(NC, N, F_OUT)
    return Hacc


# speedup vs baseline: 8.9657x; 1.0005x over previous
import functools

import jax
import jax.numpy as jnp
from jax import lax
from jax.experimental import pallas as pl
from jax.experimental.pallas import tpu as pltpu
from jax.experimental.pallas import tpu_sc as plsc

N = 10000
E = 320000

_BCAST_DNUMS = lax.GatherDimensionNumbers(
    offset_dims=(), collapsed_slice_dims=(0,), start_index_map=(0,))


def _lane_bcast(vec, lane):
    idx = jnp.full((16, 1), lane, jnp.int32)
    return lax.gather(vec, idx, _BCAST_DNUMS, slice_sizes=(1,),
                      mode=lax.GatherScatterMode.PROMISE_IN_BOUNDS)
F_IN = 128
F_OUT = 128
PERIODS = 12

NC, NS = 2, 16
NTILES = NC * NS
CHUNK = 128
SUP = 8
N_PAD = 10240
ROWS_PER_TILE = N_PAD // NS
WB = 128
CPT = 80
NSUP = CPT // SUP
E_PAD = NTILES * CPT * CHUNK
BN = 1000


F = 128
CB = F // 16


@functools.lru_cache(maxsize=None)
def _sc_prop_fn(P, G):
    mesh = plsc.VectorSubcoreMesh(core_axis_name="c", subcore_axis_name="s",
                                  num_cores=NC, num_subcores=NS)

    @functools.partial(
        pl.kernel,
        out_type=jax.ShapeDtypeStruct((P, NC, N_PAD, F), jnp.float32),
        mesh=mesh,
        scratch_types=[
            [pltpu.VMEM((SUP, 2, CHUNK), jnp.int32)] * 2,
            [pltpu.VMEM((SUP * CHUNK,), jnp.float32)] * 2,
            [pltpu.VMEM((CHUNK,), jnp.int32)] * 2,
            [pltpu.VMEM((CHUNK, F), jnp.float32)] * 2,
            pltpu.VMEM_SHARED((N_PAD, F), jnp.float32),
            [pltpu.SemaphoreType.DMA] * 2,
            [pltpu.SemaphoreType.DMA] * 2,
        ],
    )
    def body(table_h, pk_h, w_h, zwb_h, out_h, pk, wsup, aidx, rows, accum,
             psem, gsem):
        c = lax.axis_index("c")
        s = lax.axis_index("s")
        tl = c * NS + s
        row0 = s * ROWS_PER_TILE

        def sup_start(su, b):
            base = tl * NSUP + su
            pltpu.async_copy(pk_h.at[pl.ds(base * SUP, SUP)], pk[b], psem[b])
            pltpu.async_copy(w_h.at[pl.ds(base * SUP * CHUNK, SUP * CHUNK)],
                             wsup[b], psem[b])

        def sup_wait(su, b):
            base = tl * NSUP + su
            pltpu.make_async_copy(pk_h.at[pl.ds(base * SUP, SUP)], pk[b],
                                  psem[b]).wait()
            pltpu.make_async_copy(w_h.at[pl.ds(base * SUP * CHUNK, SUP * CHUNK)],
                                  wsup[b], psem[b]).wait()

        def gather_start(off, u, sb, rb):
            if not G:
                return
            if P > 1:
                for g in range(CHUNK // 16):
                    sl = pl.ds(g * 16, 16)
                    aidx[rb][sl] = pk[sb][u, 0, sl] + off
                idxref = aidx[rb]
            else:
                idxref = pk[sb].at[u, 0]
            pltpu.async_copy(table_h.at[idxref], rows[rb], gsem[rb])

        def gather_wait(rb):
            if G:
                pltpu.make_async_copy(table_h.at[aidx[rb]], rows[rb],
                                      gsem[rb]).wait()

        def scale(u, sb, rb):
            def group(g, carry3):
                wvec = wsup[sb][pl.ds(u * CHUNK + g * 16, 16)]
                for e in range(16):
                    col = g * 16 + e
                    wb = _lane_bcast(wvec, e)
                    for cb in range(CB):
                        sl = pl.ds(cb * 16, 16)
                        if G:
                            rows[rb][col, sl] = rows[rb][col, sl] * wb
                        else:
                            rows[rb][col, sl] = wvec * 0.0 + wb
                return carry3
            lax.fori_loop(0, CHUNK // 16, group, 0)

        def period(p, carry):
            pltpu.sync_copy(zwb_h, accum.at[pl.ds(row0, ROWS_PER_TILE)])
            plsc.subcore_barrier()
            off = p * N

            sup_start(0, 0)
            sup_wait(0, 0)
            gather_start(off, 0, 0, 0)
            sup_start(1, 1)

            def super_iter(ss, carry2):
              for sb in range(2):
                su = ss * 2 + sb
                for u in range(SUP):
                    rb = u % 2
                    nrb = 1 - rb
                    if u + 1 < SUP:
                        gather_start(off, u + 1, sb, nrb)
                    else:
                        @pl.when(su + 1 < NSUP)
                        def _():
                            sup_wait(su + 1, 1 - sb)
                            gather_start(off, 0, 1 - sb, nrb)
                    gather_wait(rb)
                    scale(u, sb, rb)
                    pltpu.sync_copy(rows[rb], accum.at[pk[sb].at[u, 1]],
                                    add=True)

                @pl.when(su + 2 < NSUP)
                def _():
                    sup_start(su + 2, sb)
              return carry2

            lax.fori_loop(0, NSUP // 2, super_iter, 0)
            plsc.subcore_barrier()
            pltpu.sync_copy(accum.at[pl.ds(row0, ROWS_PER_TILE)],
                            out_h.at[p, c, pl.ds(row0, ROWS_PER_TILE)])
            plsc.subcore_barrier()
            return carry

        lax.fori_loop(0, P, period, 0)

    return body


def _sc_prop(P, table, packed, w1d):
    zwb = jnp.zeros((ROWS_PER_TILE, F), jnp.float32)
    if table is None:
        dummy = jnp.zeros((8, F), jnp.float32)
        out = _sc_prop_fn(P, False)(dummy, packed, w1d, zwb)
    else:
        out = _sc_prop_fn(P, True)(table, packed, w1d, zwb)
    return out[:, :, :N, :]


def _dinv_body(degp_ref, dinv_ref):
    deg = degp_ref[0] + degp_ref[1] + 1.0
    dinv_ref[...] = lax.rsqrt(deg)


def _tc_dinv(degp):
    return pl.pallas_call(
        _dinv_body,
        grid=(N // BN,),
        in_specs=[pl.BlockSpec((2, BN, F), lambda nb: (0, nb, 0))],
        out_specs=pl.BlockSpec((BN, F), lambda nb: (nb, 0)),
        out_shape=jax.ShapeDtypeStruct((N, F), jnp.float32),
    )(degp)


def _xs_body(xt_ref, dinv_ref, xs_ref):
    xs_ref[0] = xt_ref[0] * dinv_ref[:, 0:1]


def _tc_xs(X_t, dinv128):
    return pl.pallas_call(
        _xs_body,
        grid=(PERIODS, N // BN),
        in_specs=[
            pl.BlockSpec((1, BN, F_IN), lambda p, nb: (p, nb, 0)),
            pl.BlockSpec((BN, F), lambda p, nb: (nb, 0)),
        ],
        out_specs=pl.BlockSpec((1, BN, F_IN), lambda p, nb: (p, nb, 0)),
        out_shape=jax.ShapeDtypeStruct((PERIODS, N, F_IN), jnp.float32),
    )(X_t, dinv128)


def _pxw_body(aggx_ref, xs_ref, dinv_ref, rp_ref, wx_ref, ball_ref, pxw_ref):
    dv = dinv_ref[:, 0:1]
    r = dv * (rp_ref[0, :, 0:1] + rp_ref[1, :, 0:1] + dv)
    px = dv * (aggx_ref[0, 0] + aggx_ref[0, 1] + xs_ref[0])
    pxw_ref[0] = (jnp.dot(px, wx_ref[...], preferred_element_type=jnp.float32)
                  + r * ball_ref[...])


def _tc_pxw(aggX, Xs, dinv128, rparts, Wx, b_all):
    grid = (PERIODS, N // BN)
    return pl.pallas_call(
        _pxw_body,
        grid=grid,
        in_specs=[
            pl.BlockSpec((1, 2, BN, F_IN), lambda p, nb: (p, 0, nb, 0)),
            pl.BlockSpec((1, BN, F_IN), lambda p, nb: (p, nb, 0)),
            pl.BlockSpec((BN, F), lambda p, nb: (nb, 0)),
            pl.BlockSpec((2, BN, F), lambda p, nb: (0, nb, 0)),
            pl.BlockSpec((F_IN, 4 * F_OUT), lambda p, nb: (0, 0)),
            pl.BlockSpec((1, 4 * F_OUT), lambda p, nb: (0, 0)),
        ],
        out_specs=pl.BlockSpec((1, BN, 4 * F_OUT), lambda p, nb: (p, nb, 0)),
        out_shape=jax.ShapeDtypeStruct((PERIODS, N, 4 * F_OUT), jnp.float32),
    )(aggX, Xs, dinv128, rparts, Wx, b_all)


def _cell_body(p, pxw_ref, aggh_ref, hsprev_ref, dinv_ref, wh_ref, cprev_ref,
               haccprev_ref, att_ref, c_ref, hs_ref, hacc_ref):
    dv = dinv_ref[:, 0:1]
    ph = dv * (aggh_ref[0] + aggh_ref[1] + hsprev_ref[...])
    g_all = pxw_ref[...] + jnp.dot(ph, wh_ref[...],
                                   preferred_element_type=jnp.float32)
    i = jax.nn.sigmoid(g_all[:, 0:F_OUT])
    f = jax.nn.sigmoid(g_all[:, F_OUT:2 * F_OUT])
    g = jnp.tanh(g_all[:, 2 * F_OUT:3 * F_OUT])
    o = jax.nn.sigmoid(g_all[:, 3 * F_OUT:4 * F_OUT])
    c_new = f * cprev_ref[...] + i * g
    h = o * jnp.tanh(c_new)
    a = att_ref[0]
    ex = jnp.exp(a - jnp.max(a))
    prob = ex[p] / jnp.sum(ex)
    c_ref[...] = c_new
    hs_ref[...] = dv * h
    hacc_ref[...] = haccprev_ref[...] + prob * h


def _tc_cell(p, pxw_p, aggH, Hs_prev, dinv128, Wh, C_prev, Hacc_prev, att2):
    grid = (N // BN,)
    return pl.pallas_call(
        functools.partial(_cell_body, p),
        grid=grid,
        in_specs=[
            pl.BlockSpec((BN, 4 * F_OUT), lambda nb: (nb, 0)),
            pl.BlockSpec((2, BN, F_OUT), lambda nb: (0, nb, 0)),
            pl.BlockSpec((BN, F_OUT), lambda nb: (nb, 0)),
            pl.BlockSpec((BN, F), lambda nb: (nb, 0)),
            pl.BlockSpec((F_OUT, 4 * F_OUT), lambda nb: (0, 0)),
            pl.BlockSpec((BN, F_OUT), lambda nb: (nb, 0)),
            pl.BlockSpec((BN, F_OUT), lambda nb: (nb, 0)),
            pl.BlockSpec((1, PERIODS), lambda nb: (0, 0)),
        ],
        out_specs=[
            pl.BlockSpec((BN, F_OUT), lambda nb: (nb, 0)),
            pl.BlockSpec((BN, F_OUT), lambda nb: (nb, 0)),
            pl.BlockSpec((BN, F_OUT), lambda nb: (nb, 0)),
        ],
        out_shape=[
            jax.ShapeDtypeStruct((N, F_OUT), jnp.float32),
            jax.ShapeDtypeStruct((N, F_OUT), jnp.float32),
            jax.ShapeDtypeStruct((N, F_OUT), jnp.float32),
        ],
    )(pxw_p, aggH, Hs_prev, dinv128, Wh, C_prev, Hacc_prev, att2)


def kernel(X, edge_index, edge_weight, attention,
           W_i, W_f, W_c, W_o, b_i, b_f, b_c, b_o):
    pad = E_PAD - E
    src = jnp.concatenate([edge_index[0], jnp.zeros((pad,), jnp.int32)])
    dst = jnp.concatenate([edge_index[1], jnp.zeros((pad,), jnp.int32)])
    w = jnp.concatenate([edge_weight, jnp.zeros((pad,), jnp.float32)])
    packed = jnp.stack([src.reshape(-1, CHUNK), dst.reshape(-1, CHUNK)],
                       axis=1)

    Wx = jnp.concatenate([W_i[:F_IN], W_f[:F_IN], W_c[:F_IN], W_o[:F_IN]], axis=1)
    Wh = jnp.concatenate([W_i[F_IN:], W_f[F_IN:], W_c[F_IN:], W_o[F_IN:]], axis=1)
    b_all = jnp.concatenate([b_i, b_f, b_c, b_o]).reshape(1, 4 * F_OUT)
    X_t = jnp.transpose(X, (2, 0, 1))
    att2 = attention.reshape(1, PERIODS)

    degp = _sc_prop(1, None, packed, w).reshape(NC, N, F)

    dinv128 = _tc_dinv(degp)
    Xs = _tc_xs(X_t, dinv128)

    rparts = _sc_prop(1, dinv128, packed, w).reshape(NC, N, F)

    aggX = _sc_prop(PERIODS, Xs.reshape(PERIODS * N, F_IN), packed, w)

    PXW = _tc_pxw(aggX, Xs, dinv128, rparts, Wx, b_all)

    zeros_nf = jnp.zeros((N, F_OUT), jnp.float32)
    zeros_agg = jnp.zeros((NC, N, F_OUT), jnp.float32)
    C = zeros_nf
    Hacc = zeros_nf
    Hs = zeros_nf
    aggH = zeros_agg
    for p in range(PERIODS):
        C, Hs, Hacc = _tc_cell(p, PXW[p], aggH, Hs, dinv128, Wh, C, Hacc, att2)
        if p + 1 < PERIODS:
            aggH = _sc_prop(1, Hs, packed, w).reshape(NC, N, F_OUT)
    return Hacc
